# baseline (speedup 1.0000x reference)
import functools

import jax
import jax.numpy as jnp
from jax import lax
from jax.experimental import pallas as pl
from jax.experimental.pallas import tpu as pltpu
from jax.experimental.pallas import tpu_sc as plsc

DIM = 32
NC = 2
NS = 16
NW = NC * NS

ROWS_STEP = 16


def _tc_project_table(t4, bd):
    n4 = t4.shape[0]
    blk = 1000

    def body(x_ref, w_ref, o_ref):
        o_ref[...] = jnp.dot(x_ref[...], w_ref[...],
                             preferred_element_type=jnp.float32)

    return pl.pallas_call(
        body,
        grid=(n4 // blk,),
        in_specs=[
            pl.BlockSpec((blk, 128), lambda i: (i, 0)),
            pl.BlockSpec((128, 128), lambda i: (0, 0)),
        ],
        out_specs=pl.BlockSpec((blk, 128), lambda i: (i, 0)),
        out_shape=jax.ShapeDtypeStruct((n4, 128), jnp.float32),
    )(t4, bd)


def _sc_gather(p, idx):
    B, L = idx.shape
    rows_per_w = B // NW
    steps = rows_per_w // ROWS_STEP
    n_step = ROWS_STEP * L

    def body(p_hbm, idx_hbm, out_hbm, idx_v, rows_v, sem):
        wid = lax.axis_index("s") * NC + lax.axis_index("c")
        row0 = wid * rows_per_w

        def step(c, _):
            r = pl.multiple_of(row0 + c * ROWS_STEP, ROWS_STEP)
            pltpu.sync_copy(idx_hbm.at[pl.ds(r, ROWS_STEP)], idx_v)
            cps = [
                pltpu.async_copy(
                    p_hbm.at[idx_v.at[j]],
                    rows_v.at[pl.ds(j * L, L)],
                    sem,
                )
                for j in range(ROWS_STEP)
            ]
            for cp in cps:
                cp.wait()
            pltpu.sync_copy(rows_v,
                            out_hbm.at[pl.ds(pl.multiple_of(r * L, 8), n_step)])
            return ()

        lax.fori_loop(0, steps, step, ())

    kern = pl.kernel(
        body,
        out_type=jax.ShapeDtypeStruct((B * L, DIM), jnp.float32),
        mesh=plsc.VectorSubcoreMesh(core_axis_name="c", subcore_axis_name="s"),
        compiler_params=pltpu.CompilerParams(use_tc_tiling_on_sc=False),
        scratch_types=[
            pltpu.VMEM((ROWS_STEP, L), jnp.int32),
            pltpu.VMEM((n_step, DIM), jnp.float32),
            pltpu.SemaphoreType.DMA,
        ],
    )
    return kern(p, idx)


def kernel(indexes, table, W):
    B, L = indexes.shape
    num = table.shape[0]
    bd = jnp.kron(jnp.eye(4, dtype=W.dtype), W.T)
    p4 = _tc_project_table(table.reshape(num * DIM // 128, 128), bd)
    p = p4.reshape(num, DIM)
    out = _sc_gather(p, indexes.astype(jnp.int32))
    return out.reshape(B, L, DIM)


# speedup vs baseline: 1.5602x; 1.5602x over previous
import functools

import jax
import jax.numpy as jnp
from jax import lax
from jax.experimental import pallas as pl
from jax.experimental.pallas import tpu as pltpu
from jax.experimental.pallas import tpu_sc as plsc

DIM = 32
NC = 2
NS = 16
NW = NC * NS

ROWS_STEP = 16


def _sc_gather(table, idx):
    B, L = idx.shape
    rows_per_w = B // NW
    steps = rows_per_w // ROWS_STEP

    def body(t_hbm, idx_hbm, out_hbm, idx_v, rows_v, sem):
        wid = lax.axis_index("s") * NC + lax.axis_index("c")
        row0 = wid * rows_per_w

        def step(c, _):
            r = pl.multiple_of(row0 + c * ROWS_STEP, ROWS_STEP)
            pltpu.sync_copy(idx_hbm.at[pl.ds(r, ROWS_STEP)], idx_v)
            cps = [
                pltpu.async_copy(
                    t_hbm.at[idx_v.at[j]],
                    rows_v.at[j],
                    sem,
                )
                for j in range(ROWS_STEP)
            ]
            for cp in cps:
                cp.wait()
            pltpu.sync_copy(rows_v, out_hbm.at[pl.ds(r, ROWS_STEP)])
            return ()

        lax.fori_loop(0, steps, step, ())

    kern = pl.kernel(
        body,
        out_type=jax.ShapeDtypeStruct((B, L, DIM), jnp.float32),
        mesh=plsc.VectorSubcoreMesh(core_axis_name="c", subcore_axis_name="s"),
        compiler_params=pltpu.CompilerParams(use_tc_tiling_on_sc=False),
        scratch_types=[
            pltpu.VMEM((ROWS_STEP, L), jnp.int32),
            pltpu.VMEM((ROWS_STEP, L, DIM), jnp.float32),
            pltpu.SemaphoreType.DMA,
        ],
    )
    return kern(table, idx)


def _tc_project(x4, bd):
    n4 = x4.shape[0]
    blk = 1024

    def body(x_ref, w_ref, o_ref):
        o_ref[...] = jnp.dot(x_ref[...], w_ref[...],
                             preferred_element_type=jnp.float32)

    return pl.pallas_call(
        body,
        grid=(n4 // blk,),
        in_specs=[
            pl.BlockSpec((blk, 128), lambda i: (i, 0)),
            pl.BlockSpec((128, 128), lambda i: (0, 0)),
        ],
        out_specs=pl.BlockSpec((blk, 128), lambda i: (i, 0)),
        out_shape=jax.ShapeDtypeStruct((n4, 128), jnp.float32),
    )(x4, bd)


def kernel(indexes, table, W):
    B, L = indexes.shape
    total = B * L
    emb3 = _sc_gather(table, indexes.astype(jnp.int32))
    bd = jnp.kron(jnp.eye(4, dtype=W.dtype), W.T)
    out4 = _tc_project(emb3.reshape(total // 4, 128), bd)
    return out4.reshape(B, L, DIM)


# speedup vs baseline: 1.5779x; 1.0113x over previous
import functools

import jax
import jax.numpy as jnp
from jax import lax
from jax.experimental import pallas as pl
from jax.experimental.pallas import tpu as pltpu
from jax.experimental.pallas import tpu_sc as plsc

DIM = 32
NC = 2
NS = 16
NW = NC * NS

CHUNK = 128
K = 8


def _sc_gather(table, idx2d, total):
    n_per_w = total // NW
    rows_per_step = K * CHUNK
    steps = n_per_w // rows_per_step

    def body(table_hbm, idx_hbm, out_hbm, idx_v, rows_v, sem):
        wid = lax.axis_index("s") * NC + lax.axis_index("c")
        base = wid * n_per_w

        def step(c, _):
            off = pl.multiple_of(base + c * rows_per_step, rows_per_step)
            pltpu.sync_copy(idx_hbm.at[pl.ds(pl.multiple_of(off // CHUNK, K), K)],
                            idx_v)
            cps = [
                pltpu.async_copy(
                    table_hbm.at[idx_v.at[j]],
                    rows_v.at[pl.ds(j * CHUNK, CHUNK)],
                    sem,
                )
                for j in range(K)
            ]
            for cp in cps:
                cp.wait()
            pltpu.sync_copy(rows_v, out_hbm.at[pl.ds(off, rows_per_step)])
            return ()

        lax.fori_loop(0, steps, step, ())

    kern = pl.kernel(
        body,
        out_type=jax.ShapeDtypeStruct((total, DIM), jnp.float32),
        mesh=plsc.VectorSubcoreMesh(core_axis_name="c", subcore_axis_name="s"),
        compiler_params=pltpu.CompilerParams(use_tc_tiling_on_sc=False),
        scratch_types=[
            pltpu.VMEM((K, CHUNK), jnp.int32),
            pltpu.VMEM((rows_per_step, DIM), jnp.float32),
            pltpu.SemaphoreType.DMA,
        ],
    )
    return kern(table, idx2d)


def _tc_project(x4, bd):
    n4 = x4.shape[0]
    blk = 1024

    def body(x_ref, w_ref, o_ref):
        o_ref[...] = jnp.dot(x_ref[...], w_ref[...],
                             preferred_element_type=jnp.float32)

    return pl.pallas_call(
        body,
        grid=(n4 // blk,),
        in_specs=[
            pl.BlockSpec((blk, 128), lambda i: (i, 0)),
            pl.BlockSpec((128, 128), lambda i: (0, 0)),
        ],
        out_specs=pl.BlockSpec((blk, 128), lambda i: (i, 0)),
        out_shape=jax.ShapeDtypeStruct((n4, 128), jnp.float32),
    )(x4, bd)


def kernel(indexes, table, W):
    B, L = indexes.shape
    total = B * L
    idx2d = indexes.reshape(-1).astype(jnp.int32).reshape(total // CHUNK, CHUNK)
    emb = _sc_gather(table, idx2d, total)
    bd = jnp.kron(jnp.eye(4, dtype=W.dtype), W.T)
    out4 = _tc_project(emb.reshape(total // 4, 128), bd)
    return out4.reshape(B, L, DIM)
